# Initial kernel scaffold; baseline (speedup 1.0000x reference)
import jax
import jax.numpy as jnp
from jax.experimental import pallas as pl


def kernel(x, edge_index, W_gcn, b_gcn, W1, b1, W2, b2, W3, b3):
    raise NotImplementedError("write your pallas kernel here")


# speedup vs baseline: 27.4162x; 27.4162x over previous
import functools

import jax
import jax.numpy as jnp
from jax import lax
from jax.experimental import pallas as pl
from jax.experimental.pallas import tpu as pltpu
from jax.experimental.pallas import tpu_sc as plsc

N = 10000
D = 128
E = 320000
MID = 256

NC = 2
NS = 16
NW = NC * NS

C = 128
RPT = 80
ROWS = NW * RPT
EPAD = ROWS * C
NPAD = 10112
RZ = NPAD // NS

_sc_mesh = plsc.VectorSubcoreMesh(
    core_axis_name="c", subcore_axis_name="s", num_cores=NC, num_subcores=NS)


@functools.partial(
    pl.kernel,
    out_type=jax.ShapeDtypeStruct((NC * NPAD,), jnp.float32),
    mesh=_sc_mesh,
    scratch_types=[
        pltpu.VMEM((RPT, C), jnp.int32),
        pltpu.VMEM((C,), jnp.float32),
        pltpu.VMEM((RZ,), jnp.float32),
        pltpu.VMEM_SHARED((NPAD,), jnp.float32),
    ],
)
def _sc_degree(dst_hbm, out_hbm, dst_v, ones_v, stage_v, acc_s):
    cid = lax.axis_index("c")
    sid = lax.axis_index("s")
    wid = sid * NC + cid

    pltpu.sync_copy(dst_hbm.at[pl.ds(wid * RPT, RPT)], dst_v)

    one = jnp.full((16,), 1.0, jnp.float32)
    zero = jnp.full((16,), 0.0, jnp.float32)

    def fill1(i, carry):
        ones_v[pl.ds(i * 16, 16)] = one
        return carry
    lax.fori_loop(0, C // 16, fill1, 0)

    def fill0(i, carry):
        stage_v[pl.ds(i * 8, 16)] = zero
        return carry
    lax.fori_loop(0, (RZ - 16) // 8 + 1, fill0, 0)

    pltpu.sync_copy(stage_v, acc_s.at[pl.ds(sid * RZ, RZ)])
    plsc.subcore_barrier()

    def body(j, carry):
        pltpu.sync_copy(ones_v, acc_s.at[dst_v.at[j]], add=True)
        return carry

    lax.fori_loop(0, RPT, body, 0)
    plsc.subcore_barrier()

    pltpu.sync_copy(acc_s.at[pl.ds(sid * RZ, RZ)], stage_v)
    pltpu.sync_copy(stage_v, out_hbm.at[pl.ds(cid * NPAD + sid * RZ, RZ)])


@functools.partial(
    pl.kernel,
    out_type=jax.ShapeDtypeStruct((NC, NPAD, D), jnp.float32),
    mesh=_sc_mesh,
    scratch_types=[
        pltpu.VMEM((RPT, C), jnp.int32),
        pltpu.VMEM((RPT, C), jnp.int32),
        pltpu.VMEM((C, D), jnp.float32),
        pltpu.SemaphoreType.DMA,
        pltpu.VMEM_SHARED((NPAD, D), jnp.float32),
    ],
)
def _sc_aggregate(y_hbm, src_hbm, dst_hbm, zeros_hbm, out_hbm,
                  src_v, dst_v, rows_v, sem, acc_s):
    cid = lax.axis_index("c")
    sid = lax.axis_index("s")
    wid = sid * NC + cid

    pltpu.sync_copy(src_hbm.at[pl.ds(wid * RPT, RPT)], src_v)
    pltpu.sync_copy(dst_hbm.at[pl.ds(wid * RPT, RPT)], dst_v)

    pltpu.sync_copy(zeros_hbm, rows_v)
    zbase = sid * RZ

    def zbody(k, carry):
        pltpu.sync_copy(rows_v, acc_s.at[pl.ds(zbase + k * C, C)])
        return carry
    lax.fori_loop(0, 4, zbody, 0)
    pltpu.sync_copy(rows_v.at[pl.ds(0, RZ - 4 * C)],
                    acc_s.at[pl.ds(zbase + 4 * C, RZ - 4 * C)])
    plsc.subcore_barrier()

    def body(j, carry):
        pltpu.async_copy(y_hbm.at[src_v.at[j]], rows_v, sem).wait()
        pltpu.sync_copy(rows_v, acc_s.at[dst_v.at[j]], add=True)
        return carry

    lax.fori_loop(0, RPT, body, 0)
    plsc.subcore_barrier()

    def obody(k, carry):
        pltpu.sync_copy(acc_s.at[pl.ds(zbase + k * C, C)], rows_v)
        pltpu.sync_copy(rows_v, out_hbm.at[cid, pl.ds(zbase + k * C, C)])
        return carry
    lax.fori_loop(0, 4, obody, 0)
    pltpu.sync_copy(acc_s.at[pl.ds(zbase + 4 * C, RZ - 4 * C)],
                    rows_v.at[pl.ds(0, RZ - 4 * C)])
    pltpu.sync_copy(rows_v.at[pl.ds(0, RZ - 4 * C)],
                    out_hbm.at[cid, pl.ds(zbase + 4 * C, RZ - 4 * C)])


def _tc_scale_body(x_ref, w_ref, dp_ref, y_ref, dinv_ref):
    dinv = lax.rsqrt(dp_ref[0] + dp_ref[1] + 1.0)
    xw = lax.dot_general(
        x_ref[...], w_ref[...], (((1,), (0,)), ((), ())),
        preferred_element_type=jnp.float32,
        precision=lax.Precision.HIGHEST)
    y_ref[...] = xw * dinv
    dinv_ref[...] = dinv


def _tc_scale(x, w, deg_parts):
    blk = 1000
    grid = N // blk
    return pl.pallas_call(
        _tc_scale_body,
        grid=(grid,),
        in_specs=[
            pl.BlockSpec((blk, D), lambda i: (i, 0)),
            pl.BlockSpec((D, D), lambda i: (0, 0)),
            pl.BlockSpec((NC, blk, 1), lambda i: (0, i, 0)),
        ],
        out_specs=[
            pl.BlockSpec((blk, D), lambda i: (i, 0)),
            pl.BlockSpec((blk, 1), lambda i: (i, 0)),
        ],
        out_shape=[
            jax.ShapeDtypeStruct((N, D), jnp.float32),
            jax.ShapeDtypeStruct((N, 1), jnp.float32),
        ],
    )(x, w, deg_parts)


def _tc_head_body(p_ref, y_ref, dinv_ref, x_ref, bg_ref,
                  w1_ref, b1_ref, w2_ref, b2_ref, w3_ref, b3_ref,
                  out_ref, acc_ref):
    i = pl.program_id(0)
    s = p_ref[0] + p_ref[1] + y_ref[...]
    t = jnp.maximum(s * dinv_ref[...] + bg_ref[...], 0.0) + x_ref[...]
    ps = jnp.sum(t, axis=0, keepdims=True)

    @pl.when(i == 0)
    def _():
        acc_ref[0:1, :] = ps

    @pl.when(i > 0)
    def _():
        acc_ref[0:1, :] = acc_ref[0:1, :] + ps

    @pl.when(i == pl.num_programs(0) - 1)
    def _():
        h = acc_ref[0:1, :]
        dg = lambda a, b: lax.dot_general(
            a, b, (((1,), (0,)), ((), ())),
            preferred_element_type=jnp.float32,
            precision=lax.Precision.HIGHEST)
        h1 = jnp.maximum(dg(h, w1_ref[...]) + b1_ref[...], 0.0)
        h2 = jnp.maximum(dg(h1, w2_ref[...]) + b2_ref[...], 0.0)
        out_ref[...] = dg(h2, w3_ref[...]) + b3_ref[...]


def _tc_head(parts, y, dinv, x, b_gcn, w1, b1, w2, b2, w3, b3):
    blk = 1000
    grid = N // blk
    return pl.pallas_call(
        _tc_head_body,
        grid=(grid,),
        in_specs=[
            pl.BlockSpec((NC, blk, D), lambda i: (0, i, 0)),
            pl.BlockSpec((blk, D), lambda i: (i, 0)),
            pl.BlockSpec((blk, 1), lambda i: (i, 0)),
            pl.BlockSpec((blk, D), lambda i: (i, 0)),
            pl.BlockSpec((1, D), lambda i: (0, 0)),
            pl.BlockSpec((D, MID), lambda i: (0, 0)),
            pl.BlockSpec((1, MID), lambda i: (0, 0)),
            pl.BlockSpec((MID, MID), lambda i: (0, 0)),
            pl.BlockSpec((1, MID), lambda i: (0, 0)),
            pl.BlockSpec((MID, 1), lambda i: (0, 0)),
            pl.BlockSpec((1, 1), lambda i: (0, 0)),
        ],
        out_specs=pl.BlockSpec((1, 1), lambda i: (0, 0)),
        out_shape=jax.ShapeDtypeStruct((1, 1), jnp.float32),
        scratch_shapes=[pltpu.VMEM((8, D), jnp.float32)],
    )(parts, y, dinv, x, b_gcn, w1, b1, w2, b2, w3, b3)


def kernel(x, edge_index, W_gcn, b_gcn, W1, b1, W2, b2, W3, b3):
    src = edge_index[0]
    dst = edge_index[1]
    pad = EPAD - E
    ar = lax.iota(jnp.int32, pad)
    src_p = jnp.concatenate([src, ar % N]).reshape(ROWS, C)
    dst_p = jnp.concatenate([dst, N + (ar % 16)]).reshape(ROWS, C)

    zeros128 = jnp.zeros((C, D), jnp.float32)

    deg_parts = _sc_degree(dst_p).reshape(NC, NPAD)[:, :N, None]
    y, dinv = _tc_scale(x, W_gcn, deg_parts)
    parts = _sc_aggregate(y, src_p, dst_p, zeros128)[:, :N, :]
    out = _tc_head(parts, y, dinv, x, b_gcn.reshape(1, D),
                   W1, b1.reshape(1, MID), W2, b2.reshape(1, MID),
                   W3, b3.reshape(1, 1))
    return out.reshape(1)


# speedup vs baseline: 36.5982x; 1.3349x over previous
import functools

import jax
import jax.numpy as jnp
from jax import lax
from jax.experimental import pallas as pl
from jax.experimental.pallas import tpu as pltpu
from jax.experimental.pallas import tpu_sc as plsc

N = 10000
D = 128
E = 320000
MID = 256

NC = 2
NS = 16
NW = NC * NS

C = 128
RPT = 80
ROWS = NW * RPT
EPAD = ROWS * C
NPAD = 10112
RZ = NPAD // NS

_sc_mesh = plsc.VectorSubcoreMesh(
    core_axis_name="c", subcore_axis_name="s", num_cores=NC, num_subcores=NS)


@functools.partial(
    pl.kernel,
    out_type=jax.ShapeDtypeStruct((NC * NPAD,), jnp.float32),
    mesh=_sc_mesh,
    scratch_types=[
        pltpu.VMEM((RPT, C), jnp.int32),
        pltpu.VMEM((C,), jnp.float32),
        pltpu.VMEM((RZ,), jnp.float32),
        pltpu.VMEM_SHARED((NPAD,), jnp.float32),
    ],
)
def _sc_degree(dst_hbm, out_hbm, dst_v, ones_v, stage_v, acc_s):
    cid = lax.axis_index("c")
    sid = lax.axis_index("s")
    wid = sid * NC + cid

    pltpu.sync_copy(dst_hbm.at[pl.ds(wid * RPT, RPT)], dst_v)

    one = jnp.full((16,), 1.0, jnp.float32)
    zero = jnp.full((16,), 0.0, jnp.float32)

    def fill1(i, carry):
        ones_v[pl.ds(i * 16, 16)] = one
        return carry
    lax.fori_loop(0, C // 16, fill1, 0)

    def fill0(i, carry):
        stage_v[pl.ds(i * 8, 16)] = zero
        return carry
    lax.fori_loop(0, (RZ - 16) // 8 + 1, fill0, 0)

    pltpu.sync_copy(stage_v, acc_s.at[pl.ds(sid * RZ, RZ)])
    plsc.subcore_barrier()

    def body(j, carry):
        pltpu.sync_copy(ones_v, acc_s.at[dst_v.at[j]], add=True)
        return carry

    lax.fori_loop(0, RPT, body, 0)
    plsc.subcore_barrier()

    pltpu.sync_copy(acc_s.at[pl.ds(sid * RZ, RZ)], stage_v)
    pltpu.sync_copy(stage_v, out_hbm.at[pl.ds(cid * NPAD + sid * RZ, RZ)])


HRPT = RPT // 2


@functools.partial(
    pl.kernel,
    out_type=jax.ShapeDtypeStruct((NC, NPAD, D), jnp.float32),
    mesh=_sc_mesh,
    scratch_types=[
        pltpu.VMEM((HRPT, C), jnp.int32),
        pltpu.VMEM((HRPT, C), jnp.int32),
        pltpu.VMEM((C, D), jnp.float32),
        pltpu.VMEM((C, D), jnp.float32),
        pltpu.SemaphoreType.DMA,
        pltpu.SemaphoreType.DMA,
        pltpu.VMEM_SHARED((NPAD, D), jnp.float32),
    ],
)
def _sc_aggregate(y_hbm, src_hbm, dst_hbm, zeros_hbm, out_hbm,
                  src_v, dst_v, buf_a, buf_b, sem_a, sem_b, acc_s):
    cid = lax.axis_index("c")
    sid = lax.axis_index("s")
    wid = sid * NC + cid

    pltpu.sync_copy(zeros_hbm, buf_a)
    zbase = sid * RZ

    def zbody(k, carry):
        pltpu.sync_copy(buf_a, acc_s.at[pl.ds(zbase + k * C, C)])
        return carry
    lax.fori_loop(0, 4, zbody, 0)
    pltpu.sync_copy(buf_a.at[pl.ds(0, RZ - 4 * C)],
                    acc_s.at[pl.ds(zbase + 4 * C, RZ - 4 * C)])
    plsc.subcore_barrier()

    for p in range(2):
        base = wid * RPT + p * HRPT
        pltpu.sync_copy(src_hbm.at[pl.ds(base, HRPT)], src_v)
        pltpu.sync_copy(dst_hbm.at[pl.ds(base, HRPT)], dst_v)

        ga = pltpu.async_copy(y_hbm.at[src_v.at[0]], buf_a, sem_a)
        gb = pltpu.async_copy(y_hbm.at[src_v.at[1]], buf_b, sem_b)

        def body(j, carry):
            ca = 2 * j
            pltpu.make_async_copy(y_hbm.at[src_v.at[ca]], buf_a, sem_a).wait()
            pltpu.sync_copy(buf_a, acc_s.at[dst_v.at[ca]], add=True)
            pltpu.async_copy(y_hbm.at[src_v.at[ca + 2]], buf_a, sem_a)
            pltpu.make_async_copy(
                y_hbm.at[src_v.at[ca + 1]], buf_b, sem_b).wait()
            pltpu.sync_copy(buf_b, acc_s.at[dst_v.at[ca + 1]], add=True)
            pltpu.async_copy(y_hbm.at[src_v.at[ca + 3]], buf_b, sem_b)
            return carry

        lax.fori_loop(0, HRPT // 2 - 1, body, 0)
        pltpu.make_async_copy(
            y_hbm.at[src_v.at[HRPT - 2]], buf_a, sem_a).wait()
        pltpu.sync_copy(buf_a, acc_s.at[dst_v.at[HRPT - 2]], add=True)
        pltpu.make_async_copy(
            y_hbm.at[src_v.at[HRPT - 1]], buf_b, sem_b).wait()
        pltpu.sync_copy(buf_b, acc_s.at[dst_v.at[HRPT - 1]], add=True)

    plsc.subcore_barrier()

    def obody(k, carry):
        pltpu.sync_copy(acc_s.at[pl.ds(zbase + k * C, C)], buf_a)
        pltpu.sync_copy(buf_a, out_hbm.at[cid, pl.ds(zbase + k * C, C)])
        return carry
    lax.fori_loop(0, 4, obody, 0)
    pltpu.sync_copy(acc_s.at[pl.ds(zbase + 4 * C, RZ - 4 * C)],
                    buf_a.at[pl.ds(0, RZ - 4 * C)])
    pltpu.sync_copy(buf_a.at[pl.ds(0, RZ - 4 * C)],
                    out_hbm.at[cid, pl.ds(zbase + 4 * C, RZ - 4 * C)])


def _tc_scale_body(x_ref, w_ref, dp_ref, y_ref, dinv_ref):
    dinv = lax.rsqrt(dp_ref[0] + dp_ref[1] + 1.0)
    xw = lax.dot_general(
        x_ref[...], w_ref[...], (((1,), (0,)), ((), ())),
        preferred_element_type=jnp.float32,
        precision=lax.Precision.HIGHEST)
    y_ref[...] = xw * dinv
    dinv_ref[...] = dinv


def _tc_scale(x, w, deg_parts):
    blk = 1000
    grid = N // blk
    return pl.pallas_call(
        _tc_scale_body,
        grid=(grid,),
        in_specs=[
            pl.BlockSpec((blk, D), lambda i: (i, 0)),
            pl.BlockSpec((D, D), lambda i: (0, 0)),
            pl.BlockSpec((NC, blk, 1), lambda i: (0, i, 0)),
        ],
        out_specs=[
            pl.BlockSpec((blk, D), lambda i: (i, 0)),
            pl.BlockSpec((blk, 1), lambda i: (i, 0)),
        ],
        out_shape=[
            jax.ShapeDtypeStruct((N, D), jnp.float32),
            jax.ShapeDtypeStruct((N, 1), jnp.float32),
        ],
    )(x, w, deg_parts)


def _tc_head_body(p_ref, y_ref, dinv_ref, x_ref, bg_ref,
                  w1_ref, b1_ref, w2_ref, b2_ref, w3_ref, b3_ref,
                  out_ref, acc_ref):
    i = pl.program_id(0)
    s = p_ref[0] + p_ref[1] + y_ref[...]
    t = jnp.maximum(s * dinv_ref[...] + bg_ref[...], 0.0) + x_ref[...]
    ps = jnp.sum(t, axis=0, keepdims=True)

    @pl.when(i == 0)
    def _():
        acc_ref[0:1, :] = ps

    @pl.when(i > 0)
    def _():
        acc_ref[0:1, :] = acc_ref[0:1, :] + ps

    @pl.when(i == pl.num_programs(0) - 1)
    def _():
        h = acc_ref[0:1, :]
        dg = lambda a, b: lax.dot_general(
            a, b, (((1,), (0,)), ((), ())),
            preferred_element_type=jnp.float32,
            precision=lax.Precision.HIGHEST)
        h1 = jnp.maximum(dg(h, w1_ref[...]) + b1_ref[...], 0.0)
        h2 = jnp.maximum(dg(h1, w2_ref[...]) + b2_ref[...], 0.0)
        out_ref[...] = dg(h2, w3_ref[...]) + b3_ref[...]


def _tc_head(parts, y, dinv, x, b_gcn, w1, b1, w2, b2, w3, b3):
    blk = 1000
    grid = N // blk
    return pl.pallas_call(
        _tc_head_body,
        grid=(grid,),
        in_specs=[
            pl.BlockSpec((NC, blk, D), lambda i: (0, i, 0)),
            pl.BlockSpec((blk, D), lambda i: (i, 0)),
            pl.BlockSpec((blk, 1), lambda i: (i, 0)),
            pl.BlockSpec((blk, D), lambda i: (i, 0)),
            pl.BlockSpec((1, D), lambda i: (0, 0)),
            pl.BlockSpec((D, MID), lambda i: (0, 0)),
            pl.BlockSpec((1, MID), lambda i: (0, 0)),
            pl.BlockSpec((MID, MID), lambda i: (0, 0)),
            pl.BlockSpec((1, MID), lambda i: (0, 0)),
            pl.BlockSpec((MID, 1), lambda i: (0, 0)),
            pl.BlockSpec((1, 1), lambda i: (0, 0)),
        ],
        out_specs=pl.BlockSpec((1, 1), lambda i: (0, 0)),
        out_shape=jax.ShapeDtypeStruct((1, 1), jnp.float32),
        scratch_shapes=[pltpu.VMEM((8, D), jnp.float32)],
    )(parts, y, dinv, x, b_gcn, w1, b1, w2, b2, w3, b3)


def kernel(x, edge_index, W_gcn, b_gcn, W1, b1, W2, b2, W3, b3):
    src = edge_index[0]
    dst = edge_index[1]
    pad = EPAD - E
    ar = lax.iota(jnp.int32, pad)
    src_p = jnp.concatenate([src, ar % N]).reshape(ROWS, C)
    dst_p = jnp.concatenate([dst, N + (ar % 16)]).reshape(ROWS, C)

    zeros128 = jnp.zeros((C, D), jnp.float32)

    deg_parts = _sc_degree(dst_p).reshape(NC, NPAD)[:, :N, None]
    y, dinv = _tc_scale(x, W_gcn, deg_parts)
    parts = _sc_aggregate(y, src_p, dst_p, zeros128)[:, :N, :]
    out = _tc_head(parts, y, dinv, x, b_gcn.reshape(1, D),
                   W1, b1.reshape(1, MID), W2, b2.reshape(1, MID),
                   W3, b3.reshape(1, 1))
    return out.reshape(1)


# speedup vs baseline: 37.1096x; 1.0140x over previous
import functools

import jax
import jax.numpy as jnp
from jax import lax
from jax.experimental import pallas as pl
from jax.experimental.pallas import tpu as pltpu
from jax.experimental.pallas import tpu_sc as plsc

N = 10000
D = 128
E = 320000
MID = 256

NC = 2
NS = 16
NW = NC * NS

C = 128
RPT = 80
ROWS = NW * RPT
EPAD = ROWS * C
NPAD = 10112
RZ = NPAD // NS

_sc_mesh = plsc.VectorSubcoreMesh(
    core_axis_name="c", subcore_axis_name="s", num_cores=NC, num_subcores=NS)


MROWS = 2496
TROWS = ROWS - MROWS


@functools.partial(
    pl.kernel,
    out_type=jax.ShapeDtypeStruct((NC * NPAD,), jnp.float32),
    mesh=_sc_mesh,
    scratch_types=[
        pltpu.VMEM((RPT, C), jnp.int32),
        pltpu.VMEM((C,), jnp.float32),
        pltpu.VMEM((RZ,), jnp.float32),
        pltpu.SemaphoreType.DMA,
        pltpu.VMEM_SHARED((NPAD,), jnp.float32),
    ],
)
def _sc_degree(dst_hbm, dstt_hbm, out_hbm, dst_v, ones_v, stage_v, sem, acc_s):
    cid = lax.axis_index("c")
    sid = lax.axis_index("s")
    wid = sid * NC + cid

    @pl.when(wid != NW - 1)
    def _():
        pltpu.sync_copy(dst_hbm.at[pl.ds(wid * RPT, RPT)], dst_v)

    @pl.when(wid == NW - 1)
    def _():
        pltpu.sync_copy(dst_hbm.at[pl.ds(MROWS - 16, 16)],
                        dst_v.at[pl.ds(0, 16)])
        pltpu.sync_copy(dstt_hbm, dst_v.at[pl.ds(16, TROWS)])

    one = jnp.full((16,), 1.0, jnp.float32)
    zero = jnp.full((16,), 0.0, jnp.float32)

    def fill1(i, carry):
        ones_v[pl.ds(i * 16, 16)] = one
        return carry
    lax.fori_loop(0, C // 16, fill1, 0)

    def fill0(i, carry):
        stage_v[pl.ds(i * 8, 16)] = zero
        return carry
    lax.fori_loop(0, (RZ - 16) // 8 + 1, fill0, 0)

    pltpu.sync_copy(stage_v, acc_s.at[pl.ds(sid * RZ, RZ)])
    plsc.subcore_barrier()

    def body(j, carry):
        pltpu.async_copy(ones_v, acc_s.at[dst_v.at[j]], sem, add=True)
        return carry
    lax.fori_loop(0, RPT, body, 0)

    def drain(j, carry):
        pltpu.make_async_copy(ones_v, acc_s.at[dst_v.at[0]], sem).wait()
        return carry
    lax.fori_loop(0, RPT, drain, 0)
    plsc.subcore_barrier()

    pltpu.sync_copy(acc_s.at[pl.ds(sid * RZ, RZ)], stage_v)
    pltpu.sync_copy(stage_v, out_hbm.at[pl.ds(cid * NPAD + sid * RZ, RZ)])


HRPT = RPT // 2


@functools.partial(
    pl.kernel,
    out_type=jax.ShapeDtypeStruct((NC, NPAD, D), jnp.float32),
    mesh=_sc_mesh,
    scratch_types=[
        pltpu.VMEM((HRPT, C), jnp.int32),
        pltpu.VMEM((HRPT, C), jnp.int32),
        pltpu.VMEM((C, D), jnp.float32),
        pltpu.VMEM((C, D), jnp.float32),
        pltpu.SemaphoreType.DMA,
        pltpu.SemaphoreType.DMA,
        pltpu.VMEM_SHARED((NPAD, D), jnp.float32),
    ],
)
def _sc_aggregate(y_hbm, src_hbm, srct_hbm, dst_hbm, dstt_hbm, zeros_hbm,
                  out_hbm, src_v, dst_v, buf_a, buf_b, sem_a, sem_b, acc_s):
    cid = lax.axis_index("c")
    sid = lax.axis_index("s")
    wid = sid * NC + cid

    pltpu.sync_copy(zeros_hbm, buf_a)
    zbase = sid * RZ

    def zbody(k, carry):
        pltpu.sync_copy(buf_a, acc_s.at[pl.ds(zbase + k * C, C)])
        return carry
    lax.fori_loop(0, 4, zbody, 0)
    pltpu.sync_copy(buf_a.at[pl.ds(0, RZ - 4 * C)],
                    acc_s.at[pl.ds(zbase + 4 * C, RZ - 4 * C)])
    plsc.subcore_barrier()

    for p in range(2):
        base = wid * RPT + p * HRPT

        @pl.when(wid != NW - 1)
        def _():
            pltpu.sync_copy(src_hbm.at[pl.ds(base, HRPT)], src_v)
            pltpu.sync_copy(dst_hbm.at[pl.ds(base, HRPT)], dst_v)

        if p == 0:
            @pl.when(wid == NW - 1)
            def _():
                pltpu.sync_copy(src_hbm.at[pl.ds(MROWS - 16, 16)],
                                src_v.at[pl.ds(0, 16)])
                pltpu.sync_copy(srct_hbm.at[pl.ds(0, HRPT - 16)],
                                src_v.at[pl.ds(16, HRPT - 16)])
                pltpu.sync_copy(dst_hbm.at[pl.ds(MROWS - 16, 16)],
                                dst_v.at[pl.ds(0, 16)])
                pltpu.sync_copy(dstt_hbm.at[pl.ds(0, HRPT - 16)],
                                dst_v.at[pl.ds(16, HRPT - 16)])
        else:
            @pl.when(wid == NW - 1)
            def _():
                pltpu.sync_copy(srct_hbm.at[pl.ds(HRPT - 16, HRPT)], src_v)
                pltpu.sync_copy(dstt_hbm.at[pl.ds(HRPT - 16, HRPT)], dst_v)

        ga = pltpu.async_copy(y_hbm.at[src_v.at[0]], buf_a, sem_a)
        gb = pltpu.async_copy(y_hbm.at[src_v.at[1]], buf_b, sem_b)

        def body(j, carry):
            ca = 2 * j
            pltpu.make_async_copy(y_hbm.at[src_v.at[ca]], buf_a, sem_a).wait()
            pltpu.sync_copy(buf_a, acc_s.at[dst_v.at[ca]], add=True)
            pltpu.async_copy(y_hbm.at[src_v.at[ca + 2]], buf_a, sem_a)
            pltpu.make_async_copy(
                y_hbm.at[src_v.at[ca + 1]], buf_b, sem_b).wait()
            pltpu.sync_copy(buf_b, acc_s.at[dst_v.at[ca + 1]], add=True)
            pltpu.async_copy(y_hbm.at[src_v.at[ca + 3]], buf_b, sem_b)
            return carry

        lax.fori_loop(0, HRPT // 2 - 1, body, 0)
        pltpu.make_async_copy(
            y_hbm.at[src_v.at[HRPT - 2]], buf_a, sem_a).wait()
        pltpu.sync_copy(buf_a, acc_s.at[dst_v.at[HRPT - 2]], add=True)
        pltpu.make_async_copy(
            y_hbm.at[src_v.at[HRPT - 1]], buf_b, sem_b).wait()
        pltpu.sync_copy(buf_b, acc_s.at[dst_v.at[HRPT - 1]], add=True)

    plsc.subcore_barrier()

    def obody(k, carry):
        pltpu.sync_copy(acc_s.at[pl.ds(zbase + k * C, C)], buf_a)
        pltpu.sync_copy(buf_a, out_hbm.at[cid, pl.ds(zbase + k * C, C)])
        return carry
    lax.fori_loop(0, 4, obody, 0)
    pltpu.sync_copy(acc_s.at[pl.ds(zbase + 4 * C, RZ - 4 * C)],
                    buf_a.at[pl.ds(0, RZ - 4 * C)])
    pltpu.sync_copy(buf_a.at[pl.ds(0, RZ - 4 * C)],
                    out_hbm.at[cid, pl.ds(zbase + 4 * C, RZ - 4 * C)])


def _tc_scale_body(x_ref, w_ref, dp_ref, y_ref, dinv_ref):
    dinv = lax.rsqrt(dp_ref[0] + dp_ref[1] + 1.0)
    xw = lax.dot_general(
        x_ref[...], w_ref[...], (((1,), (0,)), ((), ())),
        preferred_element_type=jnp.float32,
        precision=lax.Precision.HIGHEST)
    y_ref[...] = xw * dinv
    dinv_ref[...] = dinv


def _tc_scale(x, w, deg_parts):
    blk = 1000
    grid = N // blk
    return pl.pallas_call(
        _tc_scale_body,
        grid=(grid,),
        in_specs=[
            pl.BlockSpec((blk, D), lambda i: (i, 0)),
            pl.BlockSpec((D, D), lambda i: (0, 0)),
            pl.BlockSpec((NC, blk, 1), lambda i: (0, i, 0)),
        ],
        out_specs=[
            pl.BlockSpec((blk, D), lambda i: (i, 0)),
            pl.BlockSpec((blk, 1), lambda i: (i, 0)),
        ],
        out_shape=[
            jax.ShapeDtypeStruct((N, D), jnp.float32),
            jax.ShapeDtypeStruct((N, 1), jnp.float32),
        ],
    )(x, w, deg_parts)


def _tc_head_body(p_ref, y_ref, dinv_ref, x_ref, bg_ref,
                  w1_ref, b1_ref, w2_ref, b2_ref, w3_ref, b3_ref,
                  out_ref, acc_ref):
    i = pl.program_id(0)
    s = p_ref[0] + p_ref[1] + y_ref[...]
    t = jnp.maximum(s * dinv_ref[...] + bg_ref[...], 0.0) + x_ref[...]
    ps = jnp.sum(t, axis=0, keepdims=True)

    @pl.when(i == 0)
    def _():
        acc_ref[0:1, :] = ps

    @pl.when(i > 0)
    def _():
        acc_ref[0:1, :] = acc_ref[0:1, :] + ps

    @pl.when(i == pl.num_programs(0) - 1)
    def _():
        h = acc_ref[0:1, :]
        dg = lambda a, b: lax.dot_general(
            a, b, (((1,), (0,)), ((), ())),
            preferred_element_type=jnp.float32,
            precision=lax.Precision.HIGHEST)
        h1 = jnp.maximum(dg(h, w1_ref[...]) + b1_ref[...], 0.0)
        h2 = jnp.maximum(dg(h1, w2_ref[...]) + b2_ref[...], 0.0)
        out_ref[...] = dg(h2, w3_ref[...]) + b3_ref[...]


def _tc_head(parts, y, dinv, x, b_gcn, w1, b1, w2, b2, w3, b3):
    blk = 1000
    grid = N // blk
    return pl.pallas_call(
        _tc_head_body,
        grid=(grid,),
        in_specs=[
            pl.BlockSpec((NC, blk, D), lambda i: (0, i, 0)),
            pl.BlockSpec((blk, D), lambda i: (i, 0)),
            pl.BlockSpec((blk, 1), lambda i: (i, 0)),
            pl.BlockSpec((blk, D), lambda i: (i, 0)),
            pl.BlockSpec((1, D), lambda i: (0, 0)),
            pl.BlockSpec((D, MID), lambda i: (0, 0)),
            pl.BlockSpec((1, MID), lambda i: (0, 0)),
            pl.BlockSpec((MID, MID), lambda i: (0, 0)),
            pl.BlockSpec((1, MID), lambda i: (0, 0)),
            pl.BlockSpec((MID, 1), lambda i: (0, 0)),
            pl.BlockSpec((1, 1), lambda i: (0, 0)),
        ],
        out_specs=pl.BlockSpec((1, 1), lambda i: (0, 0)),
        out_shape=jax.ShapeDtypeStruct((1, 1), jnp.float32),
        scratch_shapes=[pltpu.VMEM((8, D), jnp.float32)],
    )(parts, y, dinv, x, b_gcn, w1, b1, w2, b2, w3, b3)


def kernel(x, edge_index, W_gcn, b_gcn, W1, b1, W2, b2, W3, b3):
    src = edge_index[0]
    dst = edge_index[1]
    ME = MROWS * C
    src_m = src[:ME].reshape(MROWS, C)
    dst_m = dst[:ME].reshape(MROWS, C)
    pad = EPAD - E
    ar = lax.iota(jnp.int32, pad)
    src_t = jnp.concatenate([src[ME:], ar % N]).reshape(TROWS, C)
    dst_t = jnp.concatenate([dst[ME:], N + (ar % 16)]).reshape(TROWS, C)

    zeros128 = jnp.zeros((C, D), jnp.float32)

    deg_parts = _sc_degree(dst_m, dst_t).reshape(NC, NPAD, 1)
    y, dinv = _tc_scale(x, W_gcn, deg_parts)
    parts = _sc_aggregate(y, src_m, src_t, dst_m, dst_t, zeros128)
    out = _tc_head(parts, y, dinv, x, b_gcn.reshape(1, D),
                   W1, b1.reshape(1, MID), W2, b2.reshape(1, MID),
                   W3, b3.reshape(1, 1))
    return out.reshape(1)
